# Initial kernel scaffold; baseline (speedup 1.0000x reference)
import jax
import jax.numpy as jnp
from jax.experimental import pallas as pl


def kernel(features, edge_index, W1l, b1, W1r, W2l, b2, W2r):
    raise NotImplementedError("write your pallas kernel here")


# speedup vs baseline: 6.1504x; 6.1504x over previous
import jax
import jax.numpy as jnp
from jax import lax
from jax.experimental import pallas as pl
from jax.experimental.pallas import tpu as pltpu
from jax.experimental.pallas import tpu_sc as plsc

N = 10000
E = 320000
DF = 128
DC = 64
NC = 2
NS = 16
NW = NC * NS
EPW = E // NW
CB = 80
NJ = EPW // CB
NPAD = 10240
RPS = NPAD // NS


def _fill2(ref, rows, cols, value):
    v = jnp.full((16,), value, jnp.float32)

    @pl.loop(0, rows)
    def _row(i):
        @pl.loop(0, cols // 16)
        def _col(k):
            ref[i, pl.ds(k * 16, 16)] = v


def _fill1(ref, n, value):
    v = jnp.full((16,), value, jnp.float32)

    @pl.loop(0, n // 16)
    def _elt(k):
        ref[pl.ds(k * 16, 16)] = v


def _make_segsum(nphase, with_cnt):
    mesh = plsc.VectorSubcoreMesh(core_axis_name="c", subcore_axis_name="s")
    out_type = [jax.ShapeDtypeStruct((NC * NPAD, DC), jnp.float32)
                for _ in range(nphase)]
    if with_cnt:
        out_type.append(jax.ShapeDtypeStruct((NC * NPAD,), jnp.float32))
    scratch = [
        pltpu.VMEM((NJ, CB), jnp.int32),
        pltpu.VMEM((NJ, CB), jnp.int32),
        pltpu.VMEM((CB, DC), jnp.float32),
        pltpu.VMEM((128, DC), jnp.float32),
        pltpu.VMEM_SHARED((NPAD, DC), jnp.float32),
        pltpu.SemaphoreType.DMA,
    ]
    if with_cnt:
        scratch += [
            pltpu.VMEM((CB,), jnp.float32),
            pltpu.VMEM((RPS,), jnp.float32),
            pltpu.VMEM_SHARED((NPAD,), jnp.float32),
        ]

    def body(*args):
        xs = args[:nphase]
        src_hbm, dst_hbm = args[nphase:nphase + 2]
        rest = args[nphase + 2:]
        outs = rest[:nphase]
        rest = rest[nphase:]
        if with_cnt:
            (cnt_hbm, src_v, dst_v, rows_v, zb, acc_sh, sem,
             ones_v, zc, cnt_sh) = rest
        else:
            src_v, dst_v, rows_v, zb, acc_sh, sem = rest
        cid = lax.axis_index("c")
        sid = lax.axis_index("s")
        wid = sid * NC + cid

        pltpu.sync_copy(src_hbm.at[wid], src_v)
        pltpu.sync_copy(dst_hbm.at[wid], dst_v)
        _fill2(zb, 128, DC, 0.0)
        if with_cnt:
            _fill1(ones_v, CB, 1.0)
            _fill1(zc, RPS, 0.0)

        for p in range(nphase):
            for t in range(RPS // 128):
                pltpu.sync_copy(
                    zb, acc_sh.at[pl.ds(sid * RPS + t * 128, 128)])
            if with_cnt and p == 0:
                pltpu.sync_copy(zc, cnt_sh.at[pl.ds(sid * RPS, RPS)])
            plsc.subcore_barrier()

            do_cnt = with_cnt and p == 0

            @pl.loop(0, NJ)
            def _chunk(j, p=p, do_cnt=do_cnt):
                pltpu.async_copy(xs[p].at[src_v.at[j]], rows_v, sem).wait()
                pltpu.sync_copy(rows_v, acc_sh.at[dst_v.at[j]], add=True)
                if do_cnt:
                    pltpu.sync_copy(ones_v, cnt_sh.at[dst_v.at[j]],
                                    add=True)

            plsc.subcore_barrier()
            pltpu.sync_copy(acc_sh.at[pl.ds(sid * RPS, RPS)],
                            outs[p].at[pl.ds(cid * NPAD + sid * RPS, RPS)])
            if with_cnt and p == 0:
                pltpu.sync_copy(
                    cnt_sh.at[pl.ds(sid * RPS, RPS)],
                    cnt_hbm.at[pl.ds(cid * NPAD + sid * RPS, RPS)])
            if p + 1 < nphase:
                plsc.subcore_barrier()

    return pl.kernel(
        body, out_type=out_type, mesh=mesh, scratch_types=scratch,
        compiler_params=pltpu.CompilerParams(use_tc_tiling_on_sc=False))


_segsum_f = _make_segsum(2, True)
_segsum_p = _make_segsum(1, False)

_BR = 1000


def _dense1(s0, s1, cntT, x, W1l_lo, W1l_hi, b1, W1r, W2l, b2, W2r):
    def body(s0_ref, s1_ref, cnt_ref, x_ref, w1ll_ref, w1lh_ref, b1_ref,
             w1r_ref, w2l_ref, b2_ref, w2r_ref, p_ref, r_ref):
        c = jnp.maximum(cnt_ref[..., 0] + cnt_ref[..., 1], 1.0)
        ci = 1.0 / c[:, None]
        m0 = (s0_ref[0] + s0_ref[1]) * ci
        m1 = (s1_ref[0] + s1_ref[1]) * ci
        t = (jnp.dot(m0, w1ll_ref[...], preferred_element_type=jnp.float32)
             + jnp.dot(m1, w1lh_ref[...], preferred_element_type=jnp.float32)
             + jnp.dot(x_ref[...], w1r_ref[...],
                       preferred_element_type=jnp.float32)
             + b1_ref[...])
        nrm = jnp.sqrt(jnp.sum(t * t, axis=1, keepdims=True))
        h = jnp.maximum(t / jnp.maximum(nrm, 1e-12), 0.0)
        p_ref[...] = jnp.dot(h, w2l_ref[...],
                             preferred_element_type=jnp.float32)
        r_ref[...] = (jnp.dot(h, w2r_ref[...],
                              preferred_element_type=jnp.float32)
                      + b2_ref[...])

    return pl.pallas_call(
        body,
        grid=(N // _BR,),
        in_specs=[
            pl.BlockSpec((2, _BR, DC), lambda i: (0, i, 0)),
            pl.BlockSpec((2, _BR, DC), lambda i: (0, i, 0)),
            pl.BlockSpec((_BR, 2), lambda i: (i, 0)),
            pl.BlockSpec((_BR, DF), lambda i: (i, 0)),
            pl.BlockSpec((DC, DF), lambda i: (0, 0)),
            pl.BlockSpec((DC, DF), lambda i: (0, 0)),
            pl.BlockSpec((1, DF), lambda i: (0, 0)),
            pl.BlockSpec((DF, DF), lambda i: (0, 0)),
            pl.BlockSpec((DF, DC), lambda i: (0, 0)),
            pl.BlockSpec((1, DC), lambda i: (0, 0)),
            pl.BlockSpec((DF, DC), lambda i: (0, 0)),
        ],
        out_specs=[
            pl.BlockSpec((_BR, DC), lambda i: (i, 0)),
            pl.BlockSpec((_BR, DC), lambda i: (i, 0)),
        ],
        out_shape=[
            jax.ShapeDtypeStruct((N, DC), jnp.float32),
            jax.ShapeDtypeStruct((N, DC), jnp.float32),
        ],
    )(s0, s1, cntT, x, W1l_lo, W1l_hi, b1, W1r, W2l, b2, W2r)


def _dense2(acc2, cntT, r):
    def body(acc_ref, cnt_ref, r_ref, o_ref):
        c = jnp.maximum(cnt_ref[..., 0] + cnt_ref[..., 1], 1.0)
        o = (acc_ref[0] + acc_ref[1]) / c[:, None] + r_ref[...]
        nrm = jnp.sqrt(jnp.sum(o * o, axis=1, keepdims=True))
        o = o / jnp.maximum(nrm, 1e-12)
        m = jnp.max(o, axis=1, keepdims=True)
        lse = jnp.log(jnp.sum(jnp.exp(o - m), axis=1, keepdims=True))
        o_ref[...] = o - m - lse

    return pl.pallas_call(
        body,
        grid=(N // _BR,),
        in_specs=[
            pl.BlockSpec((2, _BR, DC), lambda i: (0, i, 0)),
            pl.BlockSpec((_BR, 2), lambda i: (i, 0)),
            pl.BlockSpec((_BR, DC), lambda i: (i, 0)),
        ],
        out_specs=pl.BlockSpec((_BR, DC), lambda i: (i, 0)),
        out_shape=jax.ShapeDtypeStruct((N, DC), jnp.float32),
    )(acc2, cntT, r)


def kernel(features, edge_index, W1l, b1, W1r, W2l, b2, W2r):
    src = edge_index[0].reshape(NW, NJ, CB)
    dst = edge_index[1].reshape(NW, NJ, CB)
    x0 = features[:, :DC]
    x1 = features[:, DC:]
    s0_flat, s1_flat, cnt_flat = _segsum_f(x0, x1, src, dst)
    s0 = s0_flat.reshape(NC, NPAD, DC)[:, :N]
    s1 = s1_flat.reshape(NC, NPAD, DC)[:, :N]
    cntT = cnt_flat.reshape(NC, NPAD)[:, :N].T
    p, r = _dense1(s0, s1, cntT, features,
                   W1l[:DC], W1l[DC:], b1.reshape(1, DF), W1r,
                   W2l, b2.reshape(1, DC), W2r)
    out = _segsum_p(p, src, dst)
    acc2 = (out[0] if isinstance(out, (list, tuple)) else out).reshape(
        NC, NPAD, DC)[:, :N]
    return _dense2(acc2, cntT, r)


# speedup vs baseline: 12.3041x; 2.0005x over previous
import jax
import jax.numpy as jnp
from jax import lax
from jax.experimental import pallas as pl
from jax.experimental.pallas import tpu as pltpu
from jax.experimental.pallas import tpu_sc as plsc

N = 10000
E = 320000
DF = 128
DC = 64
NC = 2
NS = 16
NW = NC * NS
EPW = E // NW
CB = 80
NJ = EPW // CB
NPAD = 10240
RPS = NPAD // NS
NB = 5
NG = NJ // NB


def _fill2(ref, rows, cols, value):
    v = jnp.full((16,), value, jnp.float32)

    @pl.loop(0, rows)
    def _row(i):
        @pl.loop(0, cols // 16)
        def _col(k):
            ref[i, pl.ds(k * 16, 16)] = v


def _fill1(ref, n, value):
    v = jnp.full((16,), value, jnp.float32)

    @pl.loop(0, n // 16)
    def _elt(k):
        ref[pl.ds(k * 16, 16)] = v


def _make_segsum(nphase, with_cnt):
    mesh = plsc.VectorSubcoreMesh(core_axis_name="c", subcore_axis_name="s")
    out_type = [jax.ShapeDtypeStruct((NC * NPAD, DC), jnp.float32)
                for _ in range(nphase)]
    if with_cnt:
        out_type.append(jax.ShapeDtypeStruct((NC * NPAD,), jnp.float32))
    scratch = [
        pltpu.VMEM((NJ, CB), jnp.int32),
        pltpu.VMEM((NJ, CB), jnp.int32),
        pltpu.VMEM((NB, CB, DC), jnp.float32),
        pltpu.VMEM((128, DC), jnp.float32),
        pltpu.VMEM_SHARED((NPAD, DC), jnp.float32),
    ] + [pltpu.SemaphoreType.DMA for _ in range(NB)]
    if with_cnt:
        scratch += [
            pltpu.VMEM((CB,), jnp.float32),
            pltpu.VMEM((RPS,), jnp.float32),
            pltpu.VMEM_SHARED((NPAD,), jnp.float32),
        ]

    def body(*args):
        xs = args[:nphase]
        src_hbm, dst_hbm = args[nphase:nphase + 2]
        rest = args[nphase + 2:]
        outs = rest[:nphase]
        rest = rest[nphase:]
        if with_cnt:
            (cnt_hbm, src_v, dst_v, rows_v, zb, acc_sh) = rest[:6]
            sems = rest[6:6 + NB]
            ones_v, zc, cnt_sh = rest[6 + NB:]
        else:
            src_v, dst_v, rows_v, zb, acc_sh = rest[:5]
            sems = rest[5:5 + NB]
        cid = lax.axis_index("c")
        sid = lax.axis_index("s")
        wid = sid * NC + cid

        pltpu.sync_copy(src_hbm.at[wid], src_v)
        pltpu.sync_copy(dst_hbm.at[wid], dst_v)
        _fill2(zb, 128, DC, 0.0)
        if with_cnt:
            _fill1(ones_v, CB, 1.0)
            _fill1(zc, RPS, 0.0)

        for p in range(nphase):
            for t in range(RPS // 128):
                pltpu.sync_copy(
                    zb, acc_sh.at[pl.ds(sid * RPS + t * 128, 128)])
            if with_cnt and p == 0:
                pltpu.sync_copy(zc, cnt_sh.at[pl.ds(sid * RPS, RPS)])
            plsc.subcore_barrier()

            do_cnt = with_cnt and p == 0

            for b in range(NB):
                pltpu.async_copy(xs[p].at[src_v.at[b]], rows_v.at[b],
                                 sems[b])

            @pl.loop(0, NG)
            def _group(g, p=p, do_cnt=do_cnt):
                for b in range(NB):
                    j = g * NB + b
                    pltpu.make_async_copy(
                        xs[p].at[src_v.at[j]], rows_v.at[b],
                        sems[b]).wait()
                    pltpu.sync_copy(rows_v.at[b], acc_sh.at[dst_v.at[j]],
                                    add=True)
                    if do_cnt:
                        pltpu.sync_copy(ones_v, cnt_sh.at[dst_v.at[j]],
                                        add=True)

                    @pl.when(g + 1 < NG)
                    def _prefetch(b=b, g=g, p=p):
                        pltpu.async_copy(
                            xs[p].at[src_v.at[(g + 1) * NB + b]],
                            rows_v.at[b], sems[b])

            plsc.subcore_barrier()
            pltpu.sync_copy(acc_sh.at[pl.ds(sid * RPS, RPS)],
                            outs[p].at[pl.ds(cid * NPAD + sid * RPS, RPS)])
            if with_cnt and p == 0:
                pltpu.sync_copy(
                    cnt_sh.at[pl.ds(sid * RPS, RPS)],
                    cnt_hbm.at[pl.ds(cid * NPAD + sid * RPS, RPS)])
            if p + 1 < nphase:
                plsc.subcore_barrier()

    return pl.kernel(
        body, out_type=out_type, mesh=mesh, scratch_types=scratch,
        compiler_params=pltpu.CompilerParams(use_tc_tiling_on_sc=False))


_segsum_f = _make_segsum(2, True)
_segsum_p = _make_segsum(1, False)

_BR = 1000


def _dense1(s0, s1, cntT, x, W1l_lo, W1l_hi, b1, W1r, W2l, b2, W2r):
    def body(s0_ref, s1_ref, cnt_ref, x_ref, w1ll_ref, w1lh_ref, b1_ref,
             w1r_ref, w2l_ref, b2_ref, w2r_ref, p_ref, r_ref):
        c = jnp.maximum(cnt_ref[..., 0] + cnt_ref[..., 1], 1.0)
        ci = 1.0 / c[:, None]
        m0 = (s0_ref[0] + s0_ref[1]) * ci
        m1 = (s1_ref[0] + s1_ref[1]) * ci
        t = (jnp.dot(m0, w1ll_ref[...], preferred_element_type=jnp.float32)
             + jnp.dot(m1, w1lh_ref[...], preferred_element_type=jnp.float32)
             + jnp.dot(x_ref[...], w1r_ref[...],
                       preferred_element_type=jnp.float32)
             + b1_ref[...])
        nrm = jnp.sqrt(jnp.sum(t * t, axis=1, keepdims=True))
        h = jnp.maximum(t / jnp.maximum(nrm, 1e-12), 0.0)
        p_ref[...] = jnp.dot(h, w2l_ref[...],
                             preferred_element_type=jnp.float32)
        r_ref[...] = (jnp.dot(h, w2r_ref[...],
                              preferred_element_type=jnp.float32)
                      + b2_ref[...])

    return pl.pallas_call(
        body,
        grid=(N // _BR,),
        in_specs=[
            pl.BlockSpec((2, _BR, DC), lambda i: (0, i, 0)),
            pl.BlockSpec((2, _BR, DC), lambda i: (0, i, 0)),
            pl.BlockSpec((_BR, 2), lambda i: (i, 0)),
            pl.BlockSpec((_BR, DF), lambda i: (i, 0)),
            pl.BlockSpec((DC, DF), lambda i: (0, 0)),
            pl.BlockSpec((DC, DF), lambda i: (0, 0)),
            pl.BlockSpec((1, DF), lambda i: (0, 0)),
            pl.BlockSpec((DF, DF), lambda i: (0, 0)),
            pl.BlockSpec((DF, DC), lambda i: (0, 0)),
            pl.BlockSpec((1, DC), lambda i: (0, 0)),
            pl.BlockSpec((DF, DC), lambda i: (0, 0)),
        ],
        out_specs=[
            pl.BlockSpec((_BR, DC), lambda i: (i, 0)),
            pl.BlockSpec((_BR, DC), lambda i: (i, 0)),
        ],
        out_shape=[
            jax.ShapeDtypeStruct((N, DC), jnp.float32),
            jax.ShapeDtypeStruct((N, DC), jnp.float32),
        ],
    )(s0, s1, cntT, x, W1l_lo, W1l_hi, b1, W1r, W2l, b2, W2r)


def _dense2(acc2, cntT, r):
    def body(acc_ref, cnt_ref, r_ref, o_ref):
        c = jnp.maximum(cnt_ref[..., 0] + cnt_ref[..., 1], 1.0)
        o = (acc_ref[0] + acc_ref[1]) / c[:, None] + r_ref[...]
        nrm = jnp.sqrt(jnp.sum(o * o, axis=1, keepdims=True))
        o = o / jnp.maximum(nrm, 1e-12)
        m = jnp.max(o, axis=1, keepdims=True)
        lse = jnp.log(jnp.sum(jnp.exp(o - m), axis=1, keepdims=True))
        o_ref[...] = o - m - lse

    return pl.pallas_call(
        body,
        grid=(N // _BR,),
        in_specs=[
            pl.BlockSpec((2, _BR, DC), lambda i: (0, i, 0)),
            pl.BlockSpec((_BR, 2), lambda i: (i, 0)),
            pl.BlockSpec((_BR, DC), lambda i: (i, 0)),
        ],
        out_specs=pl.BlockSpec((_BR, DC), lambda i: (i, 0)),
        out_shape=jax.ShapeDtypeStruct((N, DC), jnp.float32),
    )(acc2, cntT, r)


def kernel(features, edge_index, W1l, b1, W1r, W2l, b2, W2r):
    src = edge_index[0].reshape(NW, NJ, CB)
    dst = edge_index[1].reshape(NW, NJ, CB)
    x0 = features[:, :DC]
    x1 = features[:, DC:]
    s0_flat, s1_flat, cnt_flat = _segsum_f(x0, x1, src, dst)
    s0 = s0_flat.reshape(NC, NPAD, DC)[:, :N]
    s1 = s1_flat.reshape(NC, NPAD, DC)[:, :N]
    cntT = cnt_flat.reshape(NC, NPAD)[:, :N].T
    p, r = _dense1(s0, s1, cntT, features,
                   W1l[:DC], W1l[DC:], b1.reshape(1, DF), W1r,
                   W2l, b2.reshape(1, DC), W2r)
    out = _segsum_p(p, src, dst)
    acc2 = (out[0] if isinstance(out, (list, tuple)) else out).reshape(
        NC, NPAD, DC)[:, :N]
    return _dense2(acc2, cntT, r)


# speedup vs baseline: 13.3472x; 1.0848x over previous
import jax
import jax.numpy as jnp
from jax import lax
from jax.experimental import pallas as pl
from jax.experimental.pallas import tpu as pltpu
from jax.experimental.pallas import tpu_sc as plsc

N = 10000
E = 320000
DF = 128
DC = 64
CW = 16
NC = 2
NS = 16
NW = NC * NS
EPW = E // NW
CB = 80
NJ = EPW // CB
NB = 5
NG = NJ // NB
RPS = N // NS


def _fill2(ref, rows, cols, value):
    v = jnp.full((16,), value, jnp.float32)

    @pl.loop(0, rows)
    def _row(i):
        @pl.loop(0, cols // 16)
        def _col(k):
            ref[i, pl.ds(k * 16, 16)] = v


def _make_segsum(nphase, with_cnt):
    mesh = plsc.VectorSubcoreMesh(core_axis_name="c", subcore_axis_name="s")
    out_type = [jax.ShapeDtypeStruct((NC * N, DC), jnp.float32)
                for _ in range(nphase)]
    if with_cnt:
        out_type.append(jax.ShapeDtypeStruct((NC * N, CW), jnp.float32))
    scratch = [
        pltpu.VMEM((NJ, CB), jnp.int32),
        pltpu.VMEM((NJ, CB), jnp.int32),
        pltpu.VMEM((NB, CB, DC), jnp.float32),
        pltpu.VMEM((125, DC), jnp.float32),
        pltpu.VMEM_SHARED((N, DC), jnp.float32),
    ] + [pltpu.SemaphoreType.DMA for _ in range(NB)]
    if nphase == 2:
        scratch += [pltpu.VMEM((NJ, CB), jnp.int32)]
    if with_cnt:
        scratch += [
            pltpu.VMEM((CB, CW), jnp.float32),
            pltpu.VMEM((RPS, CW), jnp.float32),
            pltpu.VMEM_SHARED((N, CW), jnp.float32),
        ]

    def body(*args):
        x_hbm, src_hbm, dst_hbm = args[:3]
        rest = args[3:]
        outs = rest[:nphase]
        cnt_hbm = rest[nphase] if with_cnt else None
        rest = rest[nphase + (1 if with_cnt else 0):]
        src_v, dst_v, rows_v, zb, acc_sh = rest[:5]
        rest = rest[5:]
        sems = rest[:NB]
        rest = rest[NB:]
        if nphase == 2:
            gidx_v = rest[0]
            rest = rest[1:]
        else:
            gidx_v = src_v
        if with_cnt:
            ones_v, zc, cnt_sh = rest

        cid = lax.axis_index("c")
        sid = lax.axis_index("s")
        wid = sid * NC + cid

        pltpu.sync_copy(src_hbm.at[wid], src_v)
        pltpu.sync_copy(dst_hbm.at[wid], dst_v)
        _fill2(zb, 125, DC, 0.0)
        if with_cnt:
            _fill2(ones_v, CB, CW, 1.0)
            _fill2(zc, RPS, CW, 0.0)

        for p in range(nphase):
            if nphase == 2:
                off = jnp.full((16,), p, jnp.int32)

                @pl.loop(0, NJ)
                def _xf(j, off=off):
                    @pl.loop(0, CB // 16)
                    def _xf16(k, j=j, off=off):
                        s = src_v[j, pl.ds(k * 16, 16)]
                        gidx_v[j, pl.ds(k * 16, 16)] = s + s + off

            for t in range(RPS // 125):
                pltpu.sync_copy(
                    zb, acc_sh.at[pl.ds(sid * RPS + t * 125, 125)])
            if with_cnt and p == 0:
                pltpu.sync_copy(zc, cnt_sh.at[pl.ds(sid * RPS, RPS)])
            plsc.subcore_barrier()

            do_cnt = with_cnt and p == 0

            for b in range(NB):
                pltpu.async_copy(x_hbm.at[gidx_v.at[b]], rows_v.at[b],
                                 sems[b])

            @pl.loop(0, NG)
            def _group(g, p=p, do_cnt=do_cnt, gidx_v=gidx_v):
                for b in range(NB):
                    j = g * NB + b
                    pltpu.make_async_copy(
                        x_hbm.at[gidx_v.at[j]], rows_v.at[b],
                        sems[b]).wait()
                    pltpu.sync_copy(rows_v.at[b], acc_sh.at[dst_v.at[j]],
                                    add=True)
                    if do_cnt:
                        pltpu.sync_copy(ones_v, cnt_sh.at[dst_v.at[j]],
                                        add=True)

                    @pl.when(g + 1 < NG)
                    def _prefetch(b=b, g=g, gidx_v=gidx_v):
                        pltpu.async_copy(
                            x_hbm.at[gidx_v.at[(g + 1) * NB + b]],
                            rows_v.at[b], sems[b])

            plsc.subcore_barrier()
            pltpu.sync_copy(acc_sh.at[pl.ds(sid * RPS, RPS)],
                            outs[p].at[pl.ds(cid * N + sid * RPS, RPS)])
            if with_cnt and p == 0:
                pltpu.sync_copy(
                    cnt_sh.at[pl.ds(sid * RPS, RPS)],
                    cnt_hbm.at[pl.ds(cid * N + sid * RPS, RPS)])
            if p + 1 < nphase:
                plsc.subcore_barrier()

    return pl.kernel(
        body, out_type=out_type, mesh=mesh, scratch_types=scratch,
        compiler_params=pltpu.CompilerParams(use_tc_tiling_on_sc=False))


_segsum_f = _make_segsum(2, True)
_segsum_p = _make_segsum(1, False)

_BR = 1000
_NBLK = N // _BR


def _dense1(s0, s1, cnt, x, W1l_lo, W1l_hi, b1, W1r, W2l, b2, W2r):
    def body(s0a_ref, s0b_ref, s1a_ref, s1b_ref, ca_ref, cb_ref, x_ref,
             w1ll_ref, w1lh_ref, b1_ref, w1r_ref, w2l_ref, b2_ref,
             w2r_ref, p_ref, r_ref):
        c = jnp.maximum(ca_ref[..., :1] + cb_ref[..., :1], 1.0)
        ci = 1.0 / c
        m0 = (s0a_ref[...] + s0b_ref[...]) * ci
        m1 = (s1a_ref[...] + s1b_ref[...]) * ci
        t = (jnp.dot(m0, w1ll_ref[...], preferred_element_type=jnp.float32)
             + jnp.dot(m1, w1lh_ref[...], preferred_element_type=jnp.float32)
             + jnp.dot(x_ref[...], w1r_ref[...],
                       preferred_element_type=jnp.float32)
             + b1_ref[...])
        nrm = jnp.sqrt(jnp.sum(t * t, axis=1, keepdims=True))
        h = jnp.maximum(t / jnp.maximum(nrm, 1e-12), 0.0)
        p_ref[...] = jnp.dot(h, w2l_ref[...],
                             preferred_element_type=jnp.float32)
        r_ref[...] = (jnp.dot(h, w2r_ref[...],
                              preferred_element_type=jnp.float32)
                      + b2_ref[...])

    half = pl.BlockSpec((_BR, DC), lambda i: (i, 0))
    half2 = pl.BlockSpec((_BR, DC), lambda i: (i + _NBLK, 0))
    return pl.pallas_call(
        body,
        grid=(_NBLK,),
        in_specs=[
            half, half2, half, half2,
            pl.BlockSpec((_BR, CW), lambda i: (i, 0)),
            pl.BlockSpec((_BR, CW), lambda i: (i + _NBLK, 0)),
            pl.BlockSpec((_BR, DF), lambda i: (i, 0)),
            pl.BlockSpec((DC, DF), lambda i: (0, 0)),
            pl.BlockSpec((DC, DF), lambda i: (0, 0)),
            pl.BlockSpec((1, DF), lambda i: (0, 0)),
            pl.BlockSpec((DF, DF), lambda i: (0, 0)),
            pl.BlockSpec((DF, DC), lambda i: (0, 0)),
            pl.BlockSpec((1, DC), lambda i: (0, 0)),
            pl.BlockSpec((DF, DC), lambda i: (0, 0)),
        ],
        out_specs=[
            pl.BlockSpec((_BR, DC), lambda i: (i, 0)),
            pl.BlockSpec((_BR, DC), lambda i: (i, 0)),
        ],
        out_shape=[
            jax.ShapeDtypeStruct((N, DC), jnp.float32),
            jax.ShapeDtypeStruct((N, DC), jnp.float32),
        ],
    )(s0, s0, s1, s1, cnt, cnt, x, W1l_lo, W1l_hi, b1, W1r, W2l, b2, W2r)


def _dense2(acc2, cnt, r):
    def body(aa_ref, ab_ref, ca_ref, cb_ref, r_ref, o_ref):
        c = jnp.maximum(ca_ref[..., :1] + cb_ref[..., :1], 1.0)
        o = (aa_ref[...] + ab_ref[...]) / c + r_ref[...]
        nrm = jnp.sqrt(jnp.sum(o * o, axis=1, keepdims=True))
        o = o / jnp.maximum(nrm, 1e-12)
        m = jnp.max(o, axis=1, keepdims=True)
        lse = jnp.log(jnp.sum(jnp.exp(o - m), axis=1, keepdims=True))
        o_ref[...] = o - m - lse

    half = pl.BlockSpec((_BR, DC), lambda i: (i, 0))
    half2 = pl.BlockSpec((_BR, DC), lambda i: (i + _NBLK, 0))
    return pl.pallas_call(
        body,
        grid=(_NBLK,),
        in_specs=[
            half, half2,
            pl.BlockSpec((_BR, CW), lambda i: (i, 0)),
            pl.BlockSpec((_BR, CW), lambda i: (i + _NBLK, 0)),
            pl.BlockSpec((_BR, DC), lambda i: (i, 0)),
        ],
        out_specs=pl.BlockSpec((_BR, DC), lambda i: (i, 0)),
        out_shape=jax.ShapeDtypeStruct((N, DC), jnp.float32),
    )(acc2, acc2, cnt, cnt, r)


def kernel(features, edge_index, W1l, b1, W1r, W2l, b2, W2r):
    src = edge_index[0].reshape(NW, NJ, CB)
    dst = edge_index[1].reshape(NW, NJ, CB)
    x2 = features.reshape(2 * N, DC)
    s0, s1, cnt = _segsum_f(x2, src, dst)
    p, r = _dense1(s0, s1, cnt, features,
                   W1l[:DC], W1l[DC:], b1.reshape(1, DF), W1r,
                   W2l, b2.reshape(1, DC), W2r)
    out = _segsum_p(p, src, dst)
    acc2 = out[0] if isinstance(out, (list, tuple)) else out
    return _dense2(acc2, cnt, r)


# speedup vs baseline: 15.0746x; 1.1294x over previous
import jax
import jax.numpy as jnp
from jax import lax
from jax.experimental import pallas as pl
from jax.experimental.pallas import tpu as pltpu
from jax.experimental.pallas import tpu_sc as plsc

N = 10000
E = 320000
DF = 128
DC = 64
CW = 16
NC = 2
NS = 16
NW = NC * NS
EPW = E // NW
CB = 80
NJ = EPW // CB
NB = 5
NG = NJ // NB
RPS = N // NS


def _fill2(ref, rows, cols, value):
    v = jnp.full((16,), value, jnp.float32)

    @pl.loop(0, rows)
    def _row(i):
        @pl.loop(0, cols // 16)
        def _col(k):
            ref[i, pl.ds(k * 16, 16)] = v


def _make_segsum(nphase, with_cnt):
    mesh = plsc.VectorSubcoreMesh(core_axis_name="c", subcore_axis_name="s")
    out_rows = NC * N if nphase == 2 else N
    out_type = [jax.ShapeDtypeStruct((out_rows, DF), jnp.float32)]
    if with_cnt:
        out_type.append(jax.ShapeDtypeStruct((N, DF), jnp.float32))
    scratch = [
        pltpu.VMEM((NJ, CB), jnp.int32),
        pltpu.VMEM((NJ, CB), jnp.int32),
        pltpu.VMEM((NB, CB, DC), jnp.float32),
        pltpu.VMEM((125, DC), jnp.float32),
        pltpu.VMEM_SHARED((N, DC), jnp.float32),
    ] + [pltpu.SemaphoreType.DMA for _ in range(NB)]
    if nphase == 2:
        scratch += [pltpu.VMEM((NJ, CB), jnp.int32)]
    if with_cnt:
        scratch += [
            pltpu.VMEM((CB, CW), jnp.float32),
            pltpu.VMEM((RPS, CW), jnp.float32),
            pltpu.VMEM_SHARED((N, CW), jnp.float32),
        ]

    def body(*args):
        x_hbm, src_hbm, dst_hbm = args[:3]
        rest = args[3:]
        out_hbm = rest[0]
        cnt_hbm = rest[1] if with_cnt else None
        rest = rest[1 + (1 if with_cnt else 0):]
        src_v, dst_v, rows_v, zb, acc_sh = rest[:5]
        rest = rest[5:]
        sems = rest[:NB]
        rest = rest[NB:]
        if nphase == 2:
            gidx_v = rest[0]
            rest = rest[1:]
        else:
            gidx_v = src_v
        if with_cnt:
            ones_v, zc, cnt_sh = rest

        cid = lax.axis_index("c")
        sid = lax.axis_index("s")
        wid = sid * NC + cid

        pltpu.sync_copy(src_hbm.at[wid], src_v)
        pltpu.sync_copy(dst_hbm.at[wid], dst_v)
        _fill2(zb, 125, DC, 0.0)
        if with_cnt:
            _fill2(ones_v, CB, CW, 1.0)
            _fill2(zc, RPS, CW, 0.0)

        for p in range(nphase):
            if nphase == 2:
                off = jnp.full((16,), p, jnp.int32)

                @pl.loop(0, NJ)
                def _xf(j, off=off):
                    @pl.loop(0, CB // 16)
                    def _xf16(k, j=j, off=off):
                        s = src_v[j, pl.ds(k * 16, 16)]
                        gidx_v[j, pl.ds(k * 16, 16)] = s + s + off

            for t in range(RPS // 125):
                pltpu.sync_copy(
                    zb, acc_sh.at[pl.ds(sid * RPS + t * 125, 125)])
            if with_cnt and p == 0:
                pltpu.sync_copy(zc, cnt_sh.at[pl.ds(sid * RPS, RPS)])
            plsc.subcore_barrier()

            do_cnt = with_cnt and p == 0

            for b in range(NB):
                pltpu.async_copy(x_hbm.at[gidx_v.at[b]], rows_v.at[b],
                                 sems[b])

            @pl.loop(0, NG)
            def _group(g, p=p, do_cnt=do_cnt, gidx_v=gidx_v):
                for b in range(NB):
                    j = g * NB + b
                    pltpu.make_async_copy(
                        x_hbm.at[gidx_v.at[j]], rows_v.at[b],
                        sems[b]).wait()
                    pltpu.sync_copy(rows_v.at[b], acc_sh.at[dst_v.at[j]],
                                    add=True)
                    if do_cnt:
                        pltpu.sync_copy(ones_v, cnt_sh.at[dst_v.at[j]],
                                        add=True)

                    @pl.when(g + 1 < NG)
                    def _prefetch(b=b, g=g, gidx_v=gidx_v):
                        pltpu.async_copy(
                            x_hbm.at[gidx_v.at[(g + 1) * NB + b]],
                            rows_v.at[b], sems[b])

            plsc.subcore_barrier()
            rows_sl = pl.ds(sid * RPS, RPS)
            if nphase == 2:
                pltpu.sync_copy(
                    acc_sh.at[rows_sl],
                    out_hbm.at[pl.ds(cid * N + sid * RPS, RPS),
                               pl.ds(p * DC, DC)])
            else:
                pltpu.sync_copy(
                    acc_sh.at[rows_sl],
                    out_hbm.at[rows_sl, pl.ds(cid * DC, DC)])
            if with_cnt and p == 0:
                pltpu.sync_copy(
                    cnt_sh.at[rows_sl],
                    cnt_hbm.at[rows_sl, pl.ds(cid * CW, CW)])
            if p + 1 < nphase:
                plsc.subcore_barrier()

    return pl.kernel(
        body, out_type=out_type, mesh=mesh, scratch_types=scratch,
        compiler_params=pltpu.CompilerParams(use_tc_tiling_on_sc=False))


_segsum_f = _make_segsum(2, True)
_segsum_p = _make_segsum(1, False)

_BR = 1000
_NBLK = N // _BR


def _dense1(s01, cnt, x, W1l, b1, W1r, W2l, b2, W2r):
    def body(sa_ref, sb_ref, c_ref, x_ref, w1l_ref, b1_ref, w1r_ref,
             w2l_ref, b2_ref, w2r_ref, p_ref, r_ref):
        cc = c_ref[...]
        c = jnp.maximum(cc[:, :1] + cc[:, CW:CW + 1], 1.0)
        m = (sa_ref[...] + sb_ref[...]) * (1.0 / c)
        t = (jnp.dot(m, w1l_ref[...], preferred_element_type=jnp.float32)
             + jnp.dot(x_ref[...], w1r_ref[...],
                       preferred_element_type=jnp.float32)
             + b1_ref[...])
        nrm = jnp.sqrt(jnp.sum(t * t, axis=1, keepdims=True))
        h = jnp.maximum(t / jnp.maximum(nrm, 1e-12), 0.0)
        p_ref[...] = jnp.dot(h, w2l_ref[...],
                             preferred_element_type=jnp.float32)
        r_ref[...] = (jnp.dot(h, w2r_ref[...],
                              preferred_element_type=jnp.float32)
                      + b2_ref[...])

    return pl.pallas_call(
        body,
        grid=(_NBLK,),
        in_specs=[
            pl.BlockSpec((_BR, DF), lambda i: (i, 0)),
            pl.BlockSpec((_BR, DF), lambda i: (i + _NBLK, 0)),
            pl.BlockSpec((_BR, DF), lambda i: (i, 0)),
            pl.BlockSpec((_BR, DF), lambda i: (i, 0)),
            pl.BlockSpec((DF, DF), lambda i: (0, 0)),
            pl.BlockSpec((1, DF), lambda i: (0, 0)),
            pl.BlockSpec((DF, DF), lambda i: (0, 0)),
            pl.BlockSpec((DF, DC), lambda i: (0, 0)),
            pl.BlockSpec((1, DC), lambda i: (0, 0)),
            pl.BlockSpec((DF, DC), lambda i: (0, 0)),
        ],
        out_specs=[
            pl.BlockSpec((_BR, DC), lambda i: (i, 0)),
            pl.BlockSpec((_BR, DC), lambda i: (i, 0)),
        ],
        out_shape=[
            jax.ShapeDtypeStruct((N, DC), jnp.float32),
            jax.ShapeDtypeStruct((N, DC), jnp.float32),
        ],
    )(s01, s01, cnt, x, W1l, b1, W1r, W2l, b2, W2r)


def _dense2(acc2, cnt, r):
    def body(a_ref, c_ref, r_ref, o_ref):
        cc = c_ref[...]
        c = jnp.maximum(cc[:, :1] + cc[:, CW:CW + 1], 1.0)
        aa = a_ref[...]
        o = (aa[:, :DC] + aa[:, DC:]) / c + r_ref[...]
        nrm = jnp.sqrt(jnp.sum(o * o, axis=1, keepdims=True))
        o = o / jnp.maximum(nrm, 1e-12)
        m = jnp.max(o, axis=1, keepdims=True)
        lse = jnp.log(jnp.sum(jnp.exp(o - m), axis=1, keepdims=True))
        o_ref[...] = o - m - lse

    return pl.pallas_call(
        body,
        grid=(_NBLK,),
        in_specs=[
            pl.BlockSpec((_BR, DF), lambda i: (i, 0)),
            pl.BlockSpec((_BR, DF), lambda i: (i, 0)),
            pl.BlockSpec((_BR, DC), lambda i: (i, 0)),
        ],
        out_specs=pl.BlockSpec((_BR, DC), lambda i: (i, 0)),
        out_shape=jax.ShapeDtypeStruct((N, DC), jnp.float32),
    )(acc2, cnt, r)


def kernel(features, edge_index, W1l, b1, W1r, W2l, b2, W2r):
    src = edge_index[0].reshape(NW, NJ, CB)
    dst = edge_index[1].reshape(NW, NJ, CB)
    x2 = features.reshape(2 * N, DC)
    s01, cnt = _segsum_f(x2, src, dst)
    p, r = _dense1(s01, cnt, features, W1l, b1.reshape(1, DF), W1r,
                   W2l, b2.reshape(1, DC), W2r)
    out = _segsum_p(p, src, dst)
    acc2 = out[0] if isinstance(out, (list, tuple)) else out
    return _dense2(acc2, cnt, r)


# speedup vs baseline: 15.5859x; 1.0339x over previous
import jax
import jax.numpy as jnp
from jax import lax
from jax.experimental import pallas as pl
from jax.experimental.pallas import tpu as pltpu
from jax.experimental.pallas import tpu_sc as plsc

N = 10000
E = 320000
DF = 128
DC = 64
CW = 16
NC = 2
NS = 16
EPT = E // NS
CB = 80
NJF = EPT // CB
NJP = NJF // NC
NB = 5
RPS = N // NS


def _fill2(ref, rows, cols, value):
    v = jnp.full((16,), value, jnp.float32)

    @pl.loop(0, rows)
    def _row(i):
        @pl.loop(0, cols // 16)
        def _col(k):
            ref[i, pl.ds(k * 16, 16)] = v


def _seg_common(acc_sh, zb, sid):
    for t in range(RPS // 25):
        pltpu.sync_copy(zb, acc_sh.at[pl.ds(sid * RPS + t * 25, 25)])


def _run_pipeline(x_hbm, gidx_v, dst_v, rows_v, sems, acc_sh, nj,
                  cnt_fn=None):
    ng = nj // NB
    for b in range(NB):
        pltpu.async_copy(x_hbm.at[gidx_v.at[b]], rows_v.at[b], sems[b])

    @pl.loop(0, ng)
    def _group(g):
        for b in range(NB):
            j = g * NB + b
            pltpu.make_async_copy(
                x_hbm.at[gidx_v.at[j]], rows_v.at[b], sems[b]).wait()
            pltpu.sync_copy(rows_v.at[b], acc_sh.at[dst_v.at[j]], add=True)
            if cnt_fn is not None:
                cnt_fn(j)

            @pl.when(g + 1 < ng)
            def _prefetch(b=b, g=g):
                pltpu.async_copy(x_hbm.at[gidx_v.at[(g + 1) * NB + b]],
                                 rows_v.at[b], sems[b])


def _make_segsum_f():
    mesh = plsc.VectorSubcoreMesh(core_axis_name="c", subcore_axis_name="s")
    out_type = [jax.ShapeDtypeStruct((N, DF), jnp.float32),
                jax.ShapeDtypeStruct((N, DF), jnp.float32)]
    scratch = [
        pltpu.VMEM((NJF, CB), jnp.int32),
        pltpu.VMEM((NJF, CB), jnp.int32),
        pltpu.VMEM((NB, CB, DC), jnp.float32),
        pltpu.VMEM((25, DC), jnp.float32),
        pltpu.VMEM_SHARED((N, DC), jnp.float32),
        pltpu.VMEM((CB, CW), jnp.float32),
        pltpu.VMEM((125, CW), jnp.float32),
        pltpu.VMEM_SHARED((N, CW), jnp.float32),
    ] + [pltpu.SemaphoreType.DMA for _ in range(NB)]

    def body(x_hbm, src_hbm, dst_hbm, out_hbm, cnt_hbm,
             src_v, dst_v, rows_v, zb, acc_sh, ones_v, zc, cnt_sh,
             *sems):
        cid = lax.axis_index("c")
        sid = lax.axis_index("s")

        pltpu.sync_copy(src_hbm.at[sid], src_v)
        pltpu.sync_copy(dst_hbm.at[sid], dst_v)
        _fill2(zb, 25, DC, 0.0)
        _fill2(ones_v, CB, CW, 1.0)
        _fill2(zc, 125, CW, 0.0)

        @pl.loop(0, NJF)
        def _xf(j):
            @pl.loop(0, CB // 16)
            def _xf16(k, j=j):
                s = src_v[j, pl.ds(k * 16, 16)]
                src_v[j, pl.ds(k * 16, 16)] = s + s + cid

        _seg_common(acc_sh, zb, sid)
        for t in range(RPS // 125):
            pltpu.sync_copy(zc, cnt_sh.at[pl.ds(sid * RPS + t * 125, 125)])
        plsc.subcore_barrier()

        def cnt_fn(j):
            @pl.when((j // NJP) == cid)
            def _():
                pltpu.sync_copy(ones_v, cnt_sh.at[dst_v.at[j]], add=True)

        _run_pipeline(x_hbm, src_v, dst_v, rows_v, sems, acc_sh, NJF,
                      cnt_fn)

        plsc.subcore_barrier()
        rows_sl = pl.ds(sid * RPS, RPS)
        pltpu.sync_copy(acc_sh.at[rows_sl],
                        out_hbm.at[rows_sl, pl.ds(cid * DC, DC)])
        pltpu.sync_copy(cnt_sh.at[rows_sl],
                        cnt_hbm.at[rows_sl, pl.ds(cid * CW, CW)])

    return pl.kernel(
        body, out_type=out_type, mesh=mesh, scratch_types=scratch,
        compiler_params=pltpu.CompilerParams(use_tc_tiling_on_sc=False))


def _make_segsum_p():
    mesh = plsc.VectorSubcoreMesh(core_axis_name="c", subcore_axis_name="s")
    out_type = [jax.ShapeDtypeStruct((N, DF), jnp.float32)]
    scratch = [
        pltpu.VMEM((NJP, CB), jnp.int32),
        pltpu.VMEM((NJP, CB), jnp.int32),
        pltpu.VMEM((NB, CB, DC), jnp.float32),
        pltpu.VMEM((25, DC), jnp.float32),
        pltpu.VMEM_SHARED((N, DC), jnp.float32),
    ] + [pltpu.SemaphoreType.DMA for _ in range(NB)]

    def body(x_hbm, src_hbm, dst_hbm, out_hbm,
             src_v, dst_v, rows_v, zb, acc_sh, *sems):
        cid = lax.axis_index("c")
        sid = lax.axis_index("s")

        pltpu.sync_copy(src_hbm.at[sid, pl.ds(cid * NJP, NJP)], src_v)
        pltpu.sync_copy(dst_hbm.at[sid, pl.ds(cid * NJP, NJP)], dst_v)
        _fill2(zb, 25, DC, 0.0)
        _seg_common(acc_sh, zb, sid)
        plsc.subcore_barrier()

        _run_pipeline(x_hbm, src_v, dst_v, rows_v, sems, acc_sh, NJP)

        plsc.subcore_barrier()
        rows_sl = pl.ds(sid * RPS, RPS)
        pltpu.sync_copy(acc_sh.at[rows_sl],
                        out_hbm.at[rows_sl, pl.ds(cid * DC, DC)])

    return pl.kernel(
        body, out_type=out_type, mesh=mesh, scratch_types=scratch,
        compiler_params=pltpu.CompilerParams(use_tc_tiling_on_sc=False))


_segsum_f = _make_segsum_f()
_segsum_p = _make_segsum_p()

_BR = 1000
_NBLK = N // _BR


def _dense1(s01, cnt, x, W1l, b1, W1r, W2l, b2, W2r):
    def body(s_ref, c_ref, x_ref, w1l_ref, b1_ref, w1r_ref,
             w2l_ref, b2_ref, w2r_ref, p_ref, r_ref):
        cc = c_ref[...]
        c = jnp.maximum(cc[:, :1] + cc[:, CW:CW + 1], 1.0)
        m = s_ref[...] * (1.0 / c)
        t = (jnp.dot(m, w1l_ref[...], preferred_element_type=jnp.float32)
             + jnp.dot(x_ref[...], w1r_ref[...],
                       preferred_element_type=jnp.float32)
             + b1_ref[...])
        nrm = jnp.sqrt(jnp.sum(t * t, axis=1, keepdims=True))
        h = jnp.maximum(t / jnp.maximum(nrm, 1e-12), 0.0)
        p_ref[...] = jnp.dot(h, w2l_ref[...],
                             preferred_element_type=jnp.float32)
        r_ref[...] = (jnp.dot(h, w2r_ref[...],
                              preferred_element_type=jnp.float32)
                      + b2_ref[...])

    return pl.pallas_call(
        body,
        grid=(_NBLK,),
        in_specs=[
            pl.BlockSpec((_BR, DF), lambda i: (i, 0)),
            pl.BlockSpec((_BR, DF), lambda i: (i, 0)),
            pl.BlockSpec((_BR, DF), lambda i: (i, 0)),
            pl.BlockSpec((DF, DF), lambda i: (0, 0)),
            pl.BlockSpec((1, DF), lambda i: (0, 0)),
            pl.BlockSpec((DF, DF), lambda i: (0, 0)),
            pl.BlockSpec((DF, DC), lambda i: (0, 0)),
            pl.BlockSpec((1, DC), lambda i: (0, 0)),
            pl.BlockSpec((DF, DC), lambda i: (0, 0)),
        ],
        out_specs=[
            pl.BlockSpec((_BR, DC), lambda i: (i, 0)),
            pl.BlockSpec((_BR, DC), lambda i: (i, 0)),
        ],
        out_shape=[
            jax.ShapeDtypeStruct((N, DC), jnp.float32),
            jax.ShapeDtypeStruct((N, DC), jnp.float32),
        ],
    )(s01, cnt, x, W1l, b1, W1r, W2l, b2, W2r)


def _dense2(acc2, cnt, r):
    def body(a_ref, c_ref, r_ref, o_ref):
        cc = c_ref[...]
        c = jnp.maximum(cc[:, :1] + cc[:, CW:CW + 1], 1.0)
        aa = a_ref[...]
        o = (aa[:, :DC] + aa[:, DC:]) / c + r_ref[...]
        nrm = jnp.sqrt(jnp.sum(o * o, axis=1, keepdims=True))
        o = o / jnp.maximum(nrm, 1e-12)
        m = jnp.max(o, axis=1, keepdims=True)
        lse = jnp.log(jnp.sum(jnp.exp(o - m), axis=1, keepdims=True))
        o_ref[...] = o - m - lse

    return pl.pallas_call(
        body,
        grid=(_NBLK,),
        in_specs=[
            pl.BlockSpec((_BR, DF), lambda i: (i, 0)),
            pl.BlockSpec((_BR, DF), lambda i: (i, 0)),
            pl.BlockSpec((_BR, DC), lambda i: (i, 0)),
        ],
        out_specs=pl.BlockSpec((_BR, DC), lambda i: (i, 0)),
        out_shape=jax.ShapeDtypeStruct((N, DC), jnp.float32),
    )(acc2, cnt, r)


def kernel(features, edge_index, W1l, b1, W1r, W2l, b2, W2r):
    src = edge_index[0].reshape(NS, NJF, CB)
    dst = edge_index[1].reshape(NS, NJF, CB)
    x2 = features.reshape(2 * N, DC)
    s01, cnt = _segsum_f(x2, src, dst)
    p, r = _dense1(s01, cnt, features, W1l, b1.reshape(1, DF), W1r,
                   W2l, b2.reshape(1, DC), W2r)
    out = _segsum_p(p, src, dst)
    acc2 = out[0] if isinstance(out, (list, tuple)) else out
    return _dense2(acc2, cnt, r)
